# baseline (speedup 1.0000x reference)
import functools

import jax
import jax.numpy as jnp
from jax import lax
from jax.experimental import pallas as pl
from jax.experimental.pallas import tpu as pltpu

_EPS = 1e-5


def _round_up(x, m):
    return (x + m - 1) // m * m


def _fused_kernel(x_ref, wt_ref, mask_ref, g_ref, b_ref, out_ref,
                  *, k, Wp, PV, CW, Cout, cnt):
    x = x_ref[0]
    NC = PV // CW

    sumv = jnp.zeros((Cout, 1), jnp.float32)
    ssqv = jnp.zeros((Cout, 1), jnp.float32)
    for c in range(NC):
        acc = jnp.zeros((Cout, CW), jnp.float32)
        for tap in range(k * k):
            off = c * CW + (tap // k) * Wp + (tap % k)
            acc = acc + jnp.dot(wt_ref[tap], x[:, off:off + CW],
                                preferred_element_type=jnp.float32)
        out_ref[0, :, c * CW:(c + 1) * CW] = acc.astype(jnp.bfloat16)
        am = acc * mask_ref[:, c * CW:(c + 1) * CW]
        sumv = sumv + jnp.sum(am, axis=-1, keepdims=True)
        ssqv = ssqv + jnp.sum(am * am, axis=-1, keepdims=True)

    mean = sumv / cnt
    var = jnp.maximum(ssqv / cnt - mean * mean, 0.0)
    scale = g_ref[...] * lax.rsqrt(var + _EPS)
    shift = b_ref[...] - mean * scale
    y = out_ref[0].astype(jnp.float32)
    out_ref[0] = jnp.maximum(y * scale + shift, 0.0).astype(jnp.bfloat16)


def kernel(x, weight, bias, gamma, beta):
    del bias
    N, Cin, H, W = x.shape
    Cout = weight.shape[0]
    k = 3
    p = k // 2

    x_pad = jnp.pad(x, ((0, 0), (0, 0), (p, p), (p, p)), mode="reflect")
    Hp, Wp = H + 2 * p, W + 2 * p
    Ho, Wo = H, W

    PV_raw = Ho * Wp
    OVR = (k - 1) * Wp + (k - 1)
    PV = _round_up(PV_raw, 128)
    L = _round_up(max(PV + OVR, Hp * Wp), 128)
    nl = PV // 128
    CW = PV
    for cand in (3, 4, 2, 5):
        if nl % cand == 0 and (PV // cand) >= 256:
            CW = PV // cand
            break

    xf = x_pad.reshape(N, Cin, Hp * Wp)
    xf = jnp.pad(xf, ((0, 0), (0, 0), (0, L - Hp * Wp))).astype(jnp.bfloat16)

    wt = jnp.transpose(weight, (2, 3, 0, 1)).reshape(k * k, Cout, Cin)
    wt = wt.astype(jnp.bfloat16)

    q = jnp.arange(PV, dtype=jnp.int32)
    mask = ((q < PV_raw) & ((q % Wp) < Wo)).astype(jnp.float32)[None, :]

    g2 = gamma.astype(jnp.float32).reshape(Cout, 1)
    b2 = beta.astype(jnp.float32).reshape(Cout, 1)

    kern = functools.partial(_fused_kernel, k=k, Wp=Wp, PV=PV, CW=CW,
                             Cout=Cout, cnt=float(Ho * Wo))
    y = pl.pallas_call(
        kern,
        out_shape=jax.ShapeDtypeStruct((N, Cout, PV), jnp.bfloat16),
        grid_spec=pltpu.PrefetchScalarGridSpec(
            num_scalar_prefetch=0,
            grid=(N,),
            in_specs=[
                pl.BlockSpec((1, Cin, L), lambda n: (n, 0, 0)),
                pl.BlockSpec((k * k, Cout, Cin), lambda n: (0, 0, 0)),
                pl.BlockSpec((1, PV), lambda n: (0, 0)),
                pl.BlockSpec((Cout, 1), lambda n: (0, 0)),
                pl.BlockSpec((Cout, 1), lambda n: (0, 0)),
            ],
            out_specs=pl.BlockSpec((1, Cout, PV), lambda n: (n, 0, 0)),
        ),
        compiler_params=pltpu.CompilerParams(
            dimension_semantics=("parallel",),
            vmem_limit_bytes=48 * 1024 * 1024),
    )(xf, wt, mask, g2, b2)

    out = y[:, :, :PV_raw].reshape(N, Cout, Ho, Wp)[:, :, :, :Wo]
    return out.astype(jnp.float32)
